# baseline (speedup 1.0000x reference)
import functools
import math

import numpy as np
import jax
import jax.numpy as jnp
from jax import lax
from jax.experimental import pallas as pl
from jax.experimental.pallas import tpu as pltpu
from jax.experimental.pallas import tpu_sc as plsc

Q_LEN = 2048
K_LEN = 2048
NUM_HEADS = 16
NUM_BUCKETS = 32
MAX_DISTANCE = 128

NUM_SHIFTS = 16
GATHERED_SHIFTS = 8
LINE_LEN = 4112
CHUNKS = LINE_LEN // 16
COPY_LEN = 4096
ROWS_PER_WORKER = Q_LEN // 2
DMA_GROUP = 16


def _bucket_shift_table() -> np.ndarray:
    d = np.arange(-(Q_LEN - 1), K_LEN, dtype=np.int32)
    half = NUM_BUCKETS // 2
    buckets = (d > 0).astype(np.int32) * half
    rp = np.abs(d)
    max_exact = half // 2
    is_small = rp < max_exact
    safe_rp = np.maximum(rp, 1).astype(np.float32)
    large = max_exact + (
        np.log(safe_rp / np.float32(max_exact))
        / np.float32(math.log(MAX_DISTANCE / max_exact))
        * np.float32(half - max_exact)
    ).astype(np.int32)
    large = np.minimum(large, half - 1)
    line = buckets + np.where(is_small, rp, large)

    j = np.arange(LINE_LEN, dtype=np.int32)
    idx = np.minimum(
        j[None, :] + np.arange(GATHERED_SHIFTS, dtype=np.int32)[:, None],
        line.shape[0] - 1)
    return line[idx].astype(np.int32)


_BUCKET_SHIFT = _bucket_shift_table()


def _sc_body(table_hbm, bidx_hbm, out_hbm, table_v, *rest):
    bidx_v = rest[:GATHERED_SHIFTS]
    lines_v = rest[GATHERED_SHIFTS:GATHERED_SHIFTS + NUM_SHIFTS]
    sem_out = rest[GATHERED_SHIFTS + NUM_SHIFTS]

    h = lax.axis_index("s")
    half = lax.axis_index("c")

    pltpu.sync_copy(table_hbm, table_v)
    for s in range(GATHERED_SHIFTS):
        pltpu.sync_copy(bidx_hbm.at[s], bidx_v[s])

    h_vec = jnp.full((16,), h, dtype=jnp.int32)
    for s in range(NUM_SHIFTS):
        off = 8 * (s // GATHERED_SHIFTS)
        n_chunks = CHUNKS if off == 0 else CHUNKS - 1
        def chunk(i, carry, s=s, off=off):
            bv = bidx_v[s % GATHERED_SHIFTS][pl.ds(i * 16 + off, 16)]
            lines_v[s][pl.ds(i * 16, 16)] = plsc.load_gather(table_v, [bv, h_vec])
            return carry
        lax.fori_loop(0, n_chunks, chunk, 0)

    q0 = half * ROWS_PER_WORKER
    def rows(g, carry):
        qb = q0 + g * DMA_GROUP
        copies = []
        for j in range(DMA_GROUP):
            q = qb + j
            s = (Q_LEN - 1 - j) % NUM_SHIFTS
            base = (Q_LEN - 1 - q) - s
            copies.append(pltpu.async_copy(
                lines_v[s].at[pl.ds(base, K_LEN)],
                out_hbm.at[h, q],
                sem_out,
            ))
        @pl.when(g > 0)
        def _drain_prev():
            for c in copies:
                c.wait()
        return carry
    lax.fori_loop(0, ROWS_PER_WORKER // DMA_GROUP, rows, 0)
    for j in range(DMA_GROUP):
        pltpu.make_async_copy(
            lines_v[0].at[pl.ds(0, K_LEN)], out_hbm.at[h, q0], sem_out
        ).wait()


@functools.partial(jax.jit, static_argnums=())
def _bias_sc(bias_table, bucket_shift):
    kern = pl.kernel(
        _sc_body,
        out_type=jax.ShapeDtypeStruct((NUM_HEADS, Q_LEN, K_LEN), jnp.float32),
        mesh=plsc.VectorSubcoreMesh(core_axis_name="c", subcore_axis_name="s"),
        scratch_types=(
            [pltpu.VMEM((NUM_BUCKETS, NUM_HEADS), jnp.float32)]
            + [pltpu.VMEM((LINE_LEN,), jnp.int32) for _ in range(GATHERED_SHIFTS)]
            + [pltpu.VMEM((LINE_LEN,), jnp.float32) for _ in range(NUM_SHIFTS)]
            + [pltpu.SemaphoreType.DMA]
        ),
        compiler_params=pltpu.CompilerParams(
            needs_layout_passes=False, use_tc_tiling_on_sc=False
        ),
    )
    return kern(bias_table, bucket_shift)


def kernel(query_length, key_length, bias_table):
    del query_length, key_length
    out = _bias_sc(bias_table, jnp.asarray(_BUCKET_SHIFT))
    return out[None]


# speedup vs baseline: 2.6022x; 2.6022x over previous
import functools
import math

import numpy as np
import jax
import jax.numpy as jnp
from jax import lax
from jax.experimental import pallas as pl
from jax.experimental.pallas import tpu as pltpu
from jax.experimental.pallas import tpu_sc as plsc

Q_LEN = 2048
K_LEN = 2048
NUM_HEADS = 16
NUM_BUCKETS = 32
MAX_DISTANCE = 128

LINE_VALID = Q_LEN + K_LEN - 1
NUM_MS = 8
LS_LEN = 4224
CHUNKS = LS_LEN // 16
NUM_S = 128
BQ = 256
GROUPS = BQ // 8


def _bucket_line() -> np.ndarray:
    d = np.arange(-(Q_LEN - 1), K_LEN, dtype=np.int32)
    half = NUM_BUCKETS // 2
    buckets = (d > 0).astype(np.int32) * half
    rp = np.abs(d)
    max_exact = half // 2
    is_small = rp < max_exact
    safe_rp = np.maximum(rp, 1).astype(np.float32)
    large = max_exact + (
        np.log(safe_rp / np.float32(max_exact))
        / np.float32(math.log(MAX_DISTANCE / max_exact))
        * np.float32(half - max_exact)
    ).astype(np.int32)
    large = np.minimum(large, half - 1)
    return (buckets + np.where(is_small, rp, large)).astype(np.int32)


def _bidx_shift_table() -> np.ndarray:
    line = _bucket_line()
    j = np.arange(LS_LEN, dtype=np.int32)
    idx = np.clip(j[None, :] - np.arange(NUM_MS, dtype=np.int32)[:, None],
                  0, LINE_VALID - 1)
    return line[idx].astype(np.int32)


_BIDX_SHIFT = _bidx_shift_table()


def _ls_body(table_hbm, bidx_hbm, out_hbm, table_v, bidx_v, line_v):
    h = lax.axis_index("s")
    half = lax.axis_index("c")

    pltpu.sync_copy(table_hbm, table_v)
    h_vec = jnp.full((16,), h, dtype=jnp.int32)
    for mm in range(NUM_MS // 2):
        m = half * (NUM_MS // 2) + mm
        pltpu.sync_copy(bidx_hbm.at[m], bidx_v)

        def chunk(i, carry):
            bv = bidx_v[pl.ds(i * 16, 16)]
            line_v[pl.ds(i * 16, 16)] = plsc.load_gather(table_v, [bv, h_vec])
            return carry
        lax.fori_loop(0, CHUNKS, chunk, 0)
        pltpu.sync_copy(line_v, out_hbm.at[h, m])


def _line_shift_sc(bias_table, bidx):
    kern = pl.kernel(
        _ls_body,
        out_type=jax.ShapeDtypeStruct((NUM_HEADS, NUM_MS, LS_LEN), jnp.float32),
        mesh=plsc.VectorSubcoreMesh(core_axis_name="c", subcore_axis_name="s"),
        scratch_types=[
            pltpu.VMEM((NUM_BUCKETS, NUM_HEADS), jnp.float32),
            pltpu.VMEM((LS_LEN,), jnp.int32),
            pltpu.VMEM((LS_LEN,), jnp.float32),
        ],
        compiler_params=pltpu.CompilerParams(
            needs_layout_passes=False, use_tc_tiling_on_sc=False
        ),
    )
    return kern(bias_table, bidx)


def _tc_body(ls_ref, out_ref, l_scr):
    qb = pl.program_id(1)

    @pl.when(qb == 0)
    def _build_table():
        ls = ls_ref[0]
        for t in range(NUM_S // 8):
            l_scr[pl.ds(8 * t, 8), :] = pltpu.roll(ls, (8 * t - 7) % LS_LEN, axis=1)

    for g in range(GROUPS):
        q_g = qb * BQ + g * 8
        a = 2040 - q_g
        s_lo = jnp.mod(-a, NUM_S)
        b = a + s_lo
        m = l_scr[pl.ds(pl.multiple_of(s_lo, 8), 8),
                  pl.ds(pl.multiple_of(b, 128), K_LEN)]
        out_ref[0, pl.ds(g * 8, 8), :] = m


@jax.jit
def _bias_kernel(bias_table, bidx):
    line_shift = _line_shift_sc(bias_table, bidx)
    out = pl.pallas_call(
        _tc_body,
        grid=(NUM_HEADS, Q_LEN // BQ),
        in_specs=[pl.BlockSpec((1, NUM_MS, LS_LEN), lambda h, qb: (h, 0, 0))],
        out_specs=pl.BlockSpec((1, BQ, K_LEN), lambda h, qb: (h, qb, 0)),
        out_shape=jax.ShapeDtypeStruct((NUM_HEADS, Q_LEN, K_LEN), jnp.float32),
        scratch_shapes=[pltpu.VMEM((NUM_S, LS_LEN), jnp.float32)],
        compiler_params=pltpu.CompilerParams(
            dimension_semantics=("arbitrary", "arbitrary"),
        ),
    )(line_shift)
    return out


def kernel(query_length, key_length, bias_table):
    del query_length, key_length
    out = _bias_kernel(bias_table, jnp.asarray(_BIDX_SHIFT))
    return out[None]


# speedup vs baseline: 3.0334x; 1.1657x over previous
import functools
import math

import numpy as np
import jax
import jax.numpy as jnp
from jax import lax
from jax.experimental import pallas as pl
from jax.experimental.pallas import tpu as pltpu
from jax.experimental.pallas import tpu_sc as plsc

Q_LEN = 2048
K_LEN = 2048
NUM_HEADS = 16
NUM_BUCKETS = 32
MAX_DISTANCE = 128

LINE_VALID = Q_LEN + K_LEN - 1
NUM_MS = 8
LS_LEN = 4224
CHUNKS = LS_LEN // 16
NUM_S = 128
BQ = 512
GROUPS = BQ // 8


def _bucket_line() -> np.ndarray:
    d = np.arange(-(Q_LEN - 1), K_LEN, dtype=np.int32)
    half = NUM_BUCKETS // 2
    buckets = (d > 0).astype(np.int32) * half
    rp = np.abs(d)
    max_exact = half // 2
    is_small = rp < max_exact
    safe_rp = np.maximum(rp, 1).astype(np.float32)
    large = max_exact + (
        np.log(safe_rp / np.float32(max_exact))
        / np.float32(math.log(MAX_DISTANCE / max_exact))
        * np.float32(half - max_exact)
    ).astype(np.int32)
    large = np.minimum(large, half - 1)
    return (buckets + np.where(is_small, rp, large)).astype(np.int32)


def _bidx_shift_table() -> np.ndarray:
    line = _bucket_line()
    j = np.arange(LS_LEN, dtype=np.int32)
    idx = np.clip(j[None, :] - np.arange(NUM_MS, dtype=np.int32)[:, None],
                  0, LINE_VALID - 1)
    return line[idx].astype(np.int32)


_BIDX_SHIFT = _bidx_shift_table()


def _ls_body(table_hbm, bidx_hbm, out_hbm, table_v, bidx_v, line_v):
    h = lax.axis_index("s")
    half = lax.axis_index("c")

    pltpu.sync_copy(table_hbm, table_v)
    h_vec = jnp.full((16,), h, dtype=jnp.int32)
    for mm in range(NUM_MS // 2):
        m = half * (NUM_MS // 2) + mm
        pltpu.sync_copy(bidx_hbm.at[m], bidx_v)

        def chunk(i, carry):
            bv = bidx_v[pl.ds(i * 16, 16)]
            line_v[pl.ds(i * 16, 16)] = plsc.load_gather(table_v, [bv, h_vec])
            return carry
        lax.fori_loop(0, CHUNKS, chunk, 0)
        pltpu.sync_copy(line_v, out_hbm.at[h, m])


def _line_shift_sc(bias_table, bidx):
    kern = pl.kernel(
        _ls_body,
        out_type=jax.ShapeDtypeStruct((NUM_HEADS, NUM_MS, LS_LEN), jnp.float32),
        mesh=plsc.VectorSubcoreMesh(core_axis_name="c", subcore_axis_name="s"),
        scratch_types=[
            pltpu.VMEM((NUM_BUCKETS, NUM_HEADS), jnp.float32),
            pltpu.VMEM((LS_LEN,), jnp.int32),
            pltpu.VMEM((LS_LEN,), jnp.float32),
        ],
        compiler_params=pltpu.CompilerParams(
            needs_layout_passes=False, use_tc_tiling_on_sc=False
        ),
    )
    return kern(bias_table, bidx)


def _tc_body(ls_ref, out_ref, l_scr):
    qb = pl.program_id(1)

    @pl.when(qb == 0)
    def _build_table():
        ls = ls_ref[0]
        for t in range(NUM_S // 8):
            l_scr[pl.ds(8 * t, 8), :] = pltpu.roll(ls, (8 * t - 7) % LS_LEN, axis=1)

    for g in range(GROUPS):
        q_g = qb * BQ + g * 8
        a = 2040 - q_g
        s_lo = jnp.mod(-a, NUM_S)
        b = a + s_lo
        m = l_scr[pl.ds(pl.multiple_of(s_lo, 8), 8),
                  pl.ds(pl.multiple_of(b, 128), K_LEN)]
        out_ref[0, pl.ds(g * 8, 8), :] = m


@jax.jit
def _bias_kernel(bias_table, bidx):
    line_shift = _line_shift_sc(bias_table, bidx)
    out = pl.pallas_call(
        _tc_body,
        grid=(NUM_HEADS, Q_LEN // BQ),
        in_specs=[pl.BlockSpec((1, NUM_MS, LS_LEN), lambda h, qb: (h, 0, 0))],
        out_specs=pl.BlockSpec((1, BQ, K_LEN), lambda h, qb: (h, qb, 0)),
        out_shape=jax.ShapeDtypeStruct((NUM_HEADS, Q_LEN, K_LEN), jnp.float32),
        scratch_shapes=[pltpu.VMEM((NUM_S, LS_LEN), jnp.float32)],
        compiler_params=pltpu.CompilerParams(
            dimension_semantics=("arbitrary", "arbitrary"),
        ),
    )(line_shift)
    return out


def kernel(query_length, key_length, bias_table):
    del query_length, key_length
    out = _bias_kernel(bias_table, jnp.asarray(_BIDX_SHIFT))
    return out[None]


# speedup vs baseline: 3.3498x; 1.1043x over previous
import functools
import math

import numpy as np
import jax
import jax.numpy as jnp
from jax import lax
from jax.experimental import pallas as pl
from jax.experimental.pallas import tpu as pltpu
from jax.experimental.pallas import tpu_sc as plsc

Q_LEN = 2048
K_LEN = 2048
NUM_HEADS = 16
NUM_BUCKETS = 32
MAX_DISTANCE = 128

LINE_VALID = Q_LEN + K_LEN - 1
NUM_MS = 8
LS_LEN = 4224
CHUNKS = LS_LEN // 16
NUM_S = 128
BQ = 1024
GROUPS = BQ // 8


def _bucket_line() -> np.ndarray:
    d = np.arange(-(Q_LEN - 1), K_LEN, dtype=np.int32)
    half = NUM_BUCKETS // 2
    buckets = (d > 0).astype(np.int32) * half
    rp = np.abs(d)
    max_exact = half // 2
    is_small = rp < max_exact
    safe_rp = np.maximum(rp, 1).astype(np.float32)
    large = max_exact + (
        np.log(safe_rp / np.float32(max_exact))
        / np.float32(math.log(MAX_DISTANCE / max_exact))
        * np.float32(half - max_exact)
    ).astype(np.int32)
    large = np.minimum(large, half - 1)
    return (buckets + np.where(is_small, rp, large)).astype(np.int32)


def _bidx_shift_table() -> np.ndarray:
    line = _bucket_line()
    j = np.arange(LS_LEN, dtype=np.int32)
    idx = np.clip(j[None, :] - np.arange(NUM_MS, dtype=np.int32)[:, None],
                  0, LINE_VALID - 1)
    return line[idx].astype(np.int32)


_BIDX_SHIFT = _bidx_shift_table()


def _ls_body(table_hbm, bidx_hbm, out_hbm, table_v, bidx_v, line_v):
    h = lax.axis_index("s")
    half = lax.axis_index("c")

    pltpu.sync_copy(table_hbm, table_v)
    h_vec = jnp.full((16,), h, dtype=jnp.int32)
    for mm in range(NUM_MS // 2):
        m = half * (NUM_MS // 2) + mm
        pltpu.sync_copy(bidx_hbm.at[m], bidx_v)

        def chunk(i, carry):
            bv = bidx_v[pl.ds(i * 16, 16)]
            line_v[pl.ds(i * 16, 16)] = plsc.load_gather(table_v, [bv, h_vec])
            return carry
        lax.fori_loop(0, CHUNKS, chunk, 0)
        pltpu.sync_copy(line_v, out_hbm.at[h, m])


def _line_shift_sc(bias_table, bidx):
    kern = pl.kernel(
        _ls_body,
        out_type=jax.ShapeDtypeStruct((NUM_HEADS, NUM_MS, LS_LEN), jnp.float32),
        mesh=plsc.VectorSubcoreMesh(core_axis_name="c", subcore_axis_name="s"),
        scratch_types=[
            pltpu.VMEM((NUM_BUCKETS, NUM_HEADS), jnp.float32),
            pltpu.VMEM((LS_LEN,), jnp.int32),
            pltpu.VMEM((LS_LEN,), jnp.float32),
        ],
        compiler_params=pltpu.CompilerParams(
            needs_layout_passes=False, use_tc_tiling_on_sc=False
        ),
    )
    return kern(bias_table, bidx)


def _tc_body(ls_ref, out_ref, l_scr):
    qb = pl.program_id(1)

    @pl.when(qb == 0)
    def _build_table():
        ls = ls_ref[0]
        for t in range(NUM_S // 8):
            l_scr[pl.ds(8 * t, 8), :] = pltpu.roll(ls, (8 * t - 7) % LS_LEN, axis=1)

    for g in range(GROUPS):
        q_g = qb * BQ + g * 8
        a = 2040 - q_g
        s_lo = jnp.mod(-a, NUM_S)
        b = a + s_lo
        m = l_scr[pl.ds(pl.multiple_of(s_lo, 8), 8),
                  pl.ds(pl.multiple_of(b, 128), K_LEN)]
        out_ref[0, pl.ds(g * 8, 8), :] = m


@jax.jit
def _bias_kernel(bias_table, bidx):
    line_shift = _line_shift_sc(bias_table, bidx)
    out = pl.pallas_call(
        _tc_body,
        grid=(NUM_HEADS, Q_LEN // BQ),
        in_specs=[pl.BlockSpec((1, NUM_MS, LS_LEN), lambda h, qb: (h, 0, 0))],
        out_specs=pl.BlockSpec((1, BQ, K_LEN), lambda h, qb: (h, qb, 0)),
        out_shape=jax.ShapeDtypeStruct((NUM_HEADS, Q_LEN, K_LEN), jnp.float32),
        scratch_shapes=[pltpu.VMEM((NUM_S, LS_LEN), jnp.float32)],
        compiler_params=pltpu.CompilerParams(
            dimension_semantics=("arbitrary", "arbitrary"),
        ),
    )(line_shift)
    return out


def kernel(query_length, key_length, bias_table):
    del query_length, key_length
    out = _bias_kernel(bias_table, jnp.asarray(_BIDX_SHIFT))
    return out[None]


# speedup vs baseline: 3.3658x; 1.0048x over previous
import functools
import math

import numpy as np
import jax
import jax.numpy as jnp
from jax import lax
from jax.experimental import pallas as pl
from jax.experimental.pallas import tpu as pltpu
from jax.experimental.pallas import tpu_sc as plsc

Q_LEN = 2048
K_LEN = 2048
NUM_HEADS = 16
NUM_BUCKETS = 32
MAX_DISTANCE = 128

LINE_VALID = Q_LEN + K_LEN - 1
NUM_MS = 8
LS_LEN = 4224
CHUNKS = LS_LEN // 16
NUM_S = 128
BQ = 2048
GROUPS = BQ // 8


def _bucket_line() -> np.ndarray:
    d = np.arange(-(Q_LEN - 1), K_LEN, dtype=np.int32)
    half = NUM_BUCKETS // 2
    buckets = (d > 0).astype(np.int32) * half
    rp = np.abs(d)
    max_exact = half // 2
    is_small = rp < max_exact
    safe_rp = np.maximum(rp, 1).astype(np.float32)
    large = max_exact + (
        np.log(safe_rp / np.float32(max_exact))
        / np.float32(math.log(MAX_DISTANCE / max_exact))
        * np.float32(half - max_exact)
    ).astype(np.int32)
    large = np.minimum(large, half - 1)
    return (buckets + np.where(is_small, rp, large)).astype(np.int32)


def _bidx_shift_table() -> np.ndarray:
    line = _bucket_line()
    j = np.arange(LS_LEN, dtype=np.int32)
    idx = np.clip(j[None, :] - np.arange(NUM_MS, dtype=np.int32)[:, None],
                  0, LINE_VALID - 1)
    return line[idx].astype(np.int32)


_BIDX_SHIFT = _bidx_shift_table()


def _ls_body(table_hbm, bidx_hbm, out_hbm, table_v, bidx_v, line_v):
    h = lax.axis_index("s")
    half = lax.axis_index("c")

    pltpu.sync_copy(table_hbm, table_v)
    h_vec = jnp.full((16,), h, dtype=jnp.int32)
    for mm in range(NUM_MS // 2):
        m = half * (NUM_MS // 2) + mm
        pltpu.sync_copy(bidx_hbm.at[m], bidx_v)

        def chunk(i, carry):
            bv = bidx_v[pl.ds(i * 16, 16)]
            line_v[pl.ds(i * 16, 16)] = plsc.load_gather(table_v, [bv, h_vec])
            return carry
        lax.fori_loop(0, CHUNKS, chunk, 0)
        pltpu.sync_copy(line_v, out_hbm.at[h, m])


def _line_shift_sc(bias_table, bidx):
    kern = pl.kernel(
        _ls_body,
        out_type=jax.ShapeDtypeStruct((NUM_HEADS, NUM_MS, LS_LEN), jnp.float32),
        mesh=plsc.VectorSubcoreMesh(core_axis_name="c", subcore_axis_name="s"),
        scratch_types=[
            pltpu.VMEM((NUM_BUCKETS, NUM_HEADS), jnp.float32),
            pltpu.VMEM((LS_LEN,), jnp.int32),
            pltpu.VMEM((LS_LEN,), jnp.float32),
        ],
        compiler_params=pltpu.CompilerParams(
            needs_layout_passes=False, use_tc_tiling_on_sc=False
        ),
    )
    return kern(bias_table, bidx)


def _tc_body(ls_ref, out_ref, l_scr):
    qb = pl.program_id(1)

    @pl.when(qb == 0)
    def _build_table():
        ls = ls_ref[0]
        for t in range(NUM_S // 8):
            l_scr[pl.ds(8 * t, 8), :] = pltpu.roll(ls, (8 * t - 7) % LS_LEN, axis=1)

    for g in range(GROUPS):
        q_g = qb * BQ + g * 8
        a = 2040 - q_g
        s_lo = jnp.mod(-a, NUM_S)
        b = a + s_lo
        m = l_scr[pl.ds(pl.multiple_of(s_lo, 8), 8),
                  pl.ds(pl.multiple_of(b, 128), K_LEN)]
        out_ref[0, pl.ds(g * 8, 8), :] = m


@jax.jit
def _bias_kernel(bias_table, bidx):
    line_shift = _line_shift_sc(bias_table, bidx)
    out = pl.pallas_call(
        _tc_body,
        grid=(NUM_HEADS, Q_LEN // BQ),
        in_specs=[pl.BlockSpec((1, NUM_MS, LS_LEN), lambda h, qb: (h, 0, 0))],
        out_specs=pl.BlockSpec((1, BQ, K_LEN), lambda h, qb: (h, qb, 0)),
        out_shape=jax.ShapeDtypeStruct((NUM_HEADS, Q_LEN, K_LEN), jnp.float32),
        scratch_shapes=[pltpu.VMEM((NUM_S, LS_LEN), jnp.float32)],
        compiler_params=pltpu.CompilerParams(
            dimension_semantics=("arbitrary", "arbitrary"),
        ),
    )(line_shift)
    return out


def kernel(query_length, key_length, bias_table):
    del query_length, key_length
    out = _bias_kernel(bias_table, jnp.asarray(_BIDX_SHIFT))
    return out[None]
